# baseline (speedup 1.0000x reference)
import functools

import jax
import jax.numpy as jnp
from jax import lax
from jax.experimental import pallas as pl
from jax.experimental.pallas import tpu as pltpu
from jax.experimental.pallas import tpu_sc as plsc

N_LIG, N_REC, D, H = 10000, 40000, 128, 128
NC, NS = 2, 16
NW = NC * NS
CB = 128
XW = 16
PW_LL = 40
PW_RL = 100
EP_LL = NW * PW_LL * CB
EP_RL = NW * PW_RL * CB
LL0, LL1 = 20, 60
RL0, RL1 = 50, 150
PW_MAX = 150
RPS = 632
ACC_ROWS = RPS * NS

_sc_mesh = plsc.VectorSubcoreMesh(core_axis_name="c", subcore_axis_name="s")
_sc_params = pltpu.CompilerParams(use_tc_tiling_on_sc=False)


def _silu(x):
    return x / (1.0 + jnp.exp(-x))


@functools.partial(
    pl.kernel,
    out_type=(
        jax.ShapeDtypeStruct((EP_LL, D), jnp.float32),
        jax.ShapeDtypeStruct((EP_LL, D), jnp.float32),
        jax.ShapeDtypeStruct((EP_LL, XW), jnp.float32),
        jax.ShapeDtypeStruct((EP_LL, XW), jnp.float32),
        jax.ShapeDtypeStruct((EP_RL, D), jnp.float32),
        jax.ShapeDtypeStruct((EP_RL, D), jnp.float32),
        jax.ShapeDtypeStruct((EP_RL, XW), jnp.float32),
        jax.ShapeDtypeStruct((EP_RL, XW), jnp.float32),
    ),
    mesh=_sc_mesh,
    compiler_params=_sc_params,
    scratch_types=[
        pltpu.VMEM((PW_MAX, CB), jnp.int32),
        pltpu.VMEM((CB, D), jnp.float32),
        pltpu.VMEM((CB, D), jnp.float32),
        pltpu.VMEM((CB, XW), jnp.float32),
        pltpu.VMEM((CB, XW), jnp.float32),
        pltpu.SemaphoreType.DMA,
        pltpu.SemaphoreType.DMA,
        pltpu.SemaphoreType.DMA,
        pltpu.SemaphoreType.DMA,
    ],
)
def _gather_kernel(h_lig, h_rec, xl, xr,
                   src_ll, dst_ll, src_rl, dst_rl,
                   hs_ll, hd_ll, xs_ll, xd_ll,
                   hs_rl, hd_rl, xs_rl, xd_rl,
                   idx_v, hb_a, hb_b, xb_a, xb_b, gsa, gsb, wsa, wsb):
    cid = lax.axis_index("c")
    sid = lax.axis_index("s")

    def pass_dir(pw, base, idx2, htab, xtab, h_out, x_out):
        pltpu.sync_copy(idx2.at[pl.ds(base, pw)], idx_v.at[pl.ds(0, pw)])

        def fire_g(s, hb, xb, sem):
            pltpu.async_copy(htab.at[idx_v.at[s]], hb, sem)
            pltpu.async_copy(xtab.at[idx_v.at[s]], xb, sem)

        def wait_g(hb, xb, sem):
            pltpu.make_async_copy(htab.at[idx_v.at[0]], hb, sem).wait()
            pltpu.make_async_copy(xtab.at[idx_v.at[0]], xb, sem).wait()

        def fire_w(s, hb, xb, sem):
            pltpu.async_copy(hb, h_out.at[pl.ds((base + s) * CB, CB)], sem)
            pltpu.async_copy(xb, x_out.at[pl.ds((base + s) * CB, CB)], sem)

        def wait_w(hb, xb, sem):
            pltpu.make_async_copy(hb, h_out.at[pl.ds(0, CB)], sem).wait()
            pltpu.make_async_copy(xb, x_out.at[pl.ds(0, CB)], sem).wait()

        fire_g(0, hb_a, xb_a, gsa)
        fire_g(1, hb_b, xb_b, gsb)

        def body(g, carry):
            wait_g(hb_a, xb_a, gsa)
            fire_w(2 * g, hb_a, xb_a, wsa)
            wait_g(hb_b, xb_b, gsb)
            fire_w(2 * g + 1, hb_b, xb_b, wsb)

            @pl.when(g < pw // 2 - 1)
            def _():
                wait_w(hb_a, xb_a, wsa)
                fire_g(2 * g + 2, hb_a, xb_a, gsa)
                wait_w(hb_b, xb_b, wsb)
                fire_g(2 * g + 3, hb_b, xb_b, gsb)

            return carry

        lax.fori_loop(0, pw // 2, body, 0)
        wait_w(hb_a, xb_a, wsa)
        wait_w(hb_b, xb_b, wsb)

    def all_passes(ll_pw, ll_base, rl_pw, rl_base):
        pass_dir(ll_pw, ll_base, src_ll, h_lig, xl, hs_ll, xs_ll)
        pass_dir(ll_pw, ll_base, dst_ll, h_lig, xl, hd_ll, xd_ll)
        pass_dir(rl_pw, rl_base, src_rl, h_rec, xr, hs_rl, xs_rl)
        pass_dir(rl_pw, rl_base, dst_rl, h_lig, xl, hd_rl, xd_rl)

    @pl.when(cid == 0)
    def _():
        all_passes(LL0, sid * LL0, RL0, sid * RL0)

    @pl.when(cid == 1)
    def _():
        all_passes(LL1, NS * LL0 + sid * LL1, RL1, NS * RL0 + sid * RL1)


_EB = 512


def _edge_mlp_body(hs_ref, hd_ref, xs_ref, xd_ref,
                   w1a, w1b, w1r, b1, w2, b2,
                   v1a, v1b, v1r, c1, w2c, c2,
                   mh_ref, mx_ref):
    hs = hs_ref[...]
    hd = hd_ref[...]
    diff = xs_ref[...] - xd_ref[...]
    d2 = jnp.sum(diff * diff, axis=1, keepdims=True)
    dij = jnp.sqrt(d2)
    xn = diff / (dij + 1e-9)
    pre = (jnp.dot(hs, w1a[...], preferred_element_type=jnp.float32)
           + jnp.dot(hd, w1b[...], preferred_element_type=jnp.float32)
           + dij * w1r[...] + b1[...])
    e1 = _silu(pre)
    mh = _silu(jnp.dot(e1, w2[...], preferred_element_type=jnp.float32) + b2[...])
    prec = (jnp.dot(hs, v1a[...], preferred_element_type=jnp.float32)
            + jnp.dot(hd, v1b[...], preferred_element_type=jnp.float32)
            + dij * v1r[...] + c1[...])
    e1c = _silu(prec)
    cc = _silu(jnp.dot(e1c, w2c[...], preferred_element_type=jnp.float32) + c2[...])
    mh_ref[...] = mh
    mx_ref[...] = cc[:, 0:1] * xn


def _edge_mlp(ep, hs, hd, xs, xd, ws):
    eb = lambda i: (i, 0)
    wb = lambda i: (0, 0)
    return pl.pallas_call(
        _edge_mlp_body,
        grid=(ep // _EB,),
        in_specs=[
            pl.BlockSpec((_EB, D), eb), pl.BlockSpec((_EB, D), eb),
            pl.BlockSpec((_EB, XW), eb), pl.BlockSpec((_EB, XW), eb),
            pl.BlockSpec((D, H), wb), pl.BlockSpec((D, H), wb),
            pl.BlockSpec((1, H), wb), pl.BlockSpec((1, H), wb),
            pl.BlockSpec((H, H), wb), pl.BlockSpec((1, H), wb),
            pl.BlockSpec((D, H), wb), pl.BlockSpec((D, H), wb),
            pl.BlockSpec((1, H), wb), pl.BlockSpec((1, H), wb),
            pl.BlockSpec((H, XW), wb), pl.BlockSpec((1, XW), wb),
        ],
        out_specs=[
            pl.BlockSpec((_EB, H), eb),
            pl.BlockSpec((_EB, XW), eb),
        ],
        out_shape=[
            jax.ShapeDtypeStruct((ep, H), jnp.float32),
            jax.ShapeDtypeStruct((ep, XW), jnp.float32),
        ],
    )(hs, hd, xs, xd, *ws)


@functools.partial(
    pl.kernel,
    out_type=(
        jax.ShapeDtypeStruct((NC, ACC_ROWS, D), jnp.float32),
        jax.ShapeDtypeStruct((NC, ACC_ROWS, XW), jnp.float32),
    ),
    mesh=_sc_mesh,
    compiler_params=_sc_params,
    scratch_types=[
        pltpu.VMEM((1, CB), jnp.int32),
        pltpu.VMEM((1, CB), jnp.int32),
        pltpu.VMEM((CB, D), jnp.float32),
        pltpu.VMEM((CB, D), jnp.float32),
        pltpu.VMEM((CB, XW), jnp.float32),
        pltpu.VMEM((CB, XW), jnp.float32),
        pltpu.VMEM_SHARED((ACC_ROWS, D), jnp.float32),
        pltpu.VMEM_SHARED((ACC_ROWS, XW), jnp.float32),
        pltpu.SemaphoreType.DMA,
        pltpu.SemaphoreType.DMA,
        pltpu.SemaphoreType.DMA,
        pltpu.SemaphoreType.DMA,
    ],
)
def _scatter_kernel(mh_ll, mx_ll, mh_rl, mx_rl, dsts_ll, dsts_rl, zh, zx,
                    part_h, part_x, idx_a, idx_b, hb_a, hb_b, xb_a, xb_b,
                    acc_h, acc_x, rsa, rsb, ssa, ssb):
    cid = lax.axis_index("c")
    sid = lax.axis_index("s")
    wid = sid * NC + cid
    r0 = sid * RPS
    pltpu.sync_copy(zh.at[pl.ds(r0, RPS)], acc_h.at[pl.ds(r0, RPS)])
    pltpu.sync_copy(zx.at[pl.ds(r0, RPS)], acc_x.at[pl.ds(r0, RPS)])
    plsc.subcore_barrier()

    def run(pw, dst3, mh_hbm, mx_hbm):
        base = wid * pw

        def fire_r(s, hb, xb, idxb, sem):
            pltpu.async_copy(mh_hbm.at[pl.ds((base + s) * CB, CB)], hb, sem)
            pltpu.async_copy(mx_hbm.at[pl.ds((base + s) * CB, CB)], xb, sem)
            pltpu.async_copy(dst3.at[wid, pl.ds(s, 1)], idxb, sem)

        def wait_r(hb, xb, idxb, sem):
            pltpu.make_async_copy(mh_hbm.at[pl.ds(0, CB)], hb, sem).wait()
            pltpu.make_async_copy(mx_hbm.at[pl.ds(0, CB)], xb, sem).wait()
            pltpu.make_async_copy(dst3.at[0, pl.ds(0, 1)], idxb, sem).wait()

        def do_sadd(hb, xb, idxb, sem):
            pltpu.async_copy(hb, acc_h.at[idxb.at[0]], sem, add=True)
            pltpu.async_copy(xb, acc_x.at[idxb.at[0]], sem, add=True)
            pltpu.make_async_copy(hb, acc_h.at[idxb.at[0]], sem).wait()
            pltpu.make_async_copy(xb, acc_x.at[idxb.at[0]], sem).wait()

        fire_r(0, hb_a, xb_a, idx_a, rsa)
        fire_r(1, hb_b, xb_b, idx_b, rsb)

        def body(g, carry):
            wait_r(hb_a, xb_a, idx_a, rsa)
            do_sadd(hb_a, xb_a, idx_a, ssa)

            @pl.when(g < pw // 2 - 1)
            def _():
                fire_r(2 * g + 2, hb_a, xb_a, idx_a, rsa)

            wait_r(hb_b, xb_b, idx_b, rsb)
            do_sadd(hb_b, xb_b, idx_b, ssb)

            @pl.when(g < pw // 2 - 1)
            def _():
                fire_r(2 * g + 3, hb_b, xb_b, idx_b, rsb)

            return carry

        lax.fori_loop(0, pw // 2, body, 0)

    run(PW_LL, dsts_ll, mh_ll, mx_ll)
    run(PW_RL, dsts_rl, mh_rl, mx_rl)
    plsc.subcore_barrier()
    pltpu.sync_copy(acc_h.at[pl.ds(r0, RPS)], part_h.at[cid, pl.ds(r0, RPS)])
    pltpu.sync_copy(acc_x.at[pl.ds(r0, RPS)], part_x.at[cid, pl.ds(r0, RPS)])


_NB = 1000


def _node_body(h_ref, ph0, ph1, xl_ref, px0, px1,
               wn1a, wn1b, bn1, wn2, bn2, ho_ref, xo_ref):
    h = h_ref[...]
    hn = ph0[...] + ph1[...]
    pre = (jnp.dot(h, wn1a[...], preferred_element_type=jnp.float32)
           + jnp.dot(hn, wn1b[...], preferred_element_type=jnp.float32)
           + bn1[...])
    m = jnp.dot(_silu(pre), wn2[...], preferred_element_type=jnp.float32) + bn2[...]
    ho_ref[...] = h + m
    xo_ref[...] = xl_ref[...] + px0[...] + px1[...]


def _node_call(h_lig, ph0, ph1, xl, px0, px1, wn1a, wn1b, bn1, wn2, bn2):
    nb = lambda i: (i, 0)
    wb = lambda i: (0, 0)
    return pl.pallas_call(
        _node_body,
        grid=(N_LIG // _NB,),
        in_specs=[
            pl.BlockSpec((_NB, D), nb), pl.BlockSpec((_NB, D), nb),
            pl.BlockSpec((_NB, D), nb),
            pl.BlockSpec((_NB, XW), nb), pl.BlockSpec((_NB, XW), nb),
            pl.BlockSpec((_NB, XW), nb),
            pl.BlockSpec((D, H), wb), pl.BlockSpec((D, H), wb),
            pl.BlockSpec((1, H), wb), pl.BlockSpec((H, D), wb),
            pl.BlockSpec((1, D), wb),
        ],
        out_specs=[
            pl.BlockSpec((_NB, D), nb),
            pl.BlockSpec((_NB, XW), nb),
        ],
        out_shape=[
            jax.ShapeDtypeStruct((N_LIG, D), jnp.float32),
            jax.ShapeDtypeStruct((N_LIG, XW), jnp.float32),
        ],
    )(h_lig, ph0, ph1, xl, px0, px1, wn1a, wn1b, bn1, wn2, bn2)


def _prep_idx(ei, ep):
    e = ei.shape[1]
    src = jnp.pad(ei[0], (0, ep - e))
    dst_g = jnp.pad(ei[1], (0, ep - e))
    dst_s = jnp.pad(ei[1], (0, ep - e), constant_values=N_LIG)
    return (src.reshape(-1, CB), dst_g.reshape(-1, CB),
            dst_s.reshape(NW, -1, CB))


def kernel(h_lig, h_rec, x_lig, x_rec, edge_index_ll, edge_index_rl,
           W1e_ll, b1e_ll, W2e_ll, b2e_ll, W1c_ll, b1c_ll, W2c_ll, b2c_ll,
           W1e_rl, b1e_rl, W2e_rl, b2e_rl, W1c_rl, b1c_rl, W2c_rl, b2c_rl,
           Wn1, bn1, Wn2, bn2):
    xl = jnp.pad(x_lig, ((0, 0), (0, XW - 3)))
    xr = jnp.pad(x_rec, ((0, 0), (0, XW - 3)))
    src_ll, dstg_ll, dsts_ll = _prep_idx(edge_index_ll, EP_LL)
    src_rl, dstg_rl, dsts_rl = _prep_idx(edge_index_rl, EP_RL)

    (hs_ll, hd_ll, xs_ll, xd_ll,
     hs_rl, hd_rl, xs_rl, xd_rl) = _gather_kernel(
        h_lig, h_rec, xl, xr, src_ll, dstg_ll, src_rl, dstg_rl)

    def ws(W1e, b1e, W2e, b2e, W1c, b1c, W2c, b2c):
        return (W1e[:D], W1e[D:2 * D], W1e[2 * D:], b1e.reshape(1, H),
                W2e, b2e.reshape(1, H),
                W1c[:D], W1c[D:2 * D], W1c[2 * D:], b1c.reshape(1, H),
                jnp.pad(W2c, ((0, 0), (0, XW - 1))),
                jnp.pad(b2c, (0, XW - 1)).reshape(1, XW))

    mh_ll, mx_ll = _edge_mlp(EP_LL, hs_ll, hd_ll, xs_ll, xd_ll,
                             ws(W1e_ll, b1e_ll, W2e_ll, b2e_ll,
                                W1c_ll, b1c_ll, W2c_ll, b2c_ll))
    mh_rl, mx_rl = _edge_mlp(EP_RL, hs_rl, hd_rl, xs_rl, xd_rl,
                             ws(W1e_rl, b1e_rl, W2e_rl, b2e_rl,
                                W1c_rl, b1c_rl, W2c_rl, b2c_rl))

    zh = jnp.zeros((ACC_ROWS, D), jnp.float32)
    zx = jnp.zeros((ACC_ROWS, XW), jnp.float32)
    part_h, part_x = _scatter_kernel(mh_ll, mx_ll, mh_rl, mx_rl,
                                     dsts_ll, dsts_rl, zh, zx)

    h_out, xo = _node_call(h_lig, part_h[0], part_h[1], xl,
                           part_x[0], part_x[1],
                           Wn1[:D], Wn1[D:], bn1.reshape(1, H),
                           Wn2, bn2.reshape(1, D))
    return (h_out, h_rec, xo[:, :3], x_rec)


# speedup vs baseline: 1.0968x; 1.0968x over previous
import functools

import jax
import jax.numpy as jnp
from jax import lax
from jax.experimental import pallas as pl
from jax.experimental.pallas import tpu as pltpu
from jax.experimental.pallas import tpu_sc as plsc

N_LIG, N_REC, D, H = 10000, 40000, 128, 128
NC, NS = 2, 16
NW = NC * NS
CB = 128
XW = 16
PW_LL = 40
PW_RL = 100
EP_LL = NW * PW_LL * CB
EP_RL = NW * PW_RL * CB
LL0, LL1 = 60, 20
RL0, RL1 = 150, 50
PW_MAX = 150
RPS = 632
ACC_ROWS = RPS * NS

_sc_mesh = plsc.VectorSubcoreMesh(core_axis_name="c", subcore_axis_name="s")
_sc_params = pltpu.CompilerParams(use_tc_tiling_on_sc=False)


def _silu(x):
    return x / (1.0 + jnp.exp(-x))


@functools.partial(
    pl.kernel,
    out_type=(
        jax.ShapeDtypeStruct((EP_LL, D), jnp.float32),
        jax.ShapeDtypeStruct((EP_LL, D), jnp.float32),
        jax.ShapeDtypeStruct((EP_LL, XW), jnp.float32),
        jax.ShapeDtypeStruct((EP_LL, XW), jnp.float32),
        jax.ShapeDtypeStruct((EP_RL, D), jnp.float32),
        jax.ShapeDtypeStruct((EP_RL, D), jnp.float32),
        jax.ShapeDtypeStruct((EP_RL, XW), jnp.float32),
        jax.ShapeDtypeStruct((EP_RL, XW), jnp.float32),
    ),
    mesh=_sc_mesh,
    compiler_params=_sc_params,
    scratch_types=[
        pltpu.VMEM((PW_MAX, CB), jnp.int32),
        pltpu.VMEM((CB, D), jnp.float32),
        pltpu.VMEM((CB, D), jnp.float32),
        pltpu.VMEM((CB, XW), jnp.float32),
        pltpu.VMEM((CB, XW), jnp.float32),
        pltpu.SemaphoreType.DMA,
        pltpu.SemaphoreType.DMA,
        pltpu.SemaphoreType.DMA,
        pltpu.SemaphoreType.DMA,
    ],
)
def _gather_kernel(h_lig, h_rec, xl, xr,
                   src_ll, dst_ll, src_rl, dst_rl,
                   hs_ll, hd_ll, xs_ll, xd_ll,
                   hs_rl, hd_rl, xs_rl, xd_rl,
                   idx_v, hb_a, hb_b, xb_a, xb_b, gsa, gsb, wsa, wsb):
    cid = lax.axis_index("c")
    sid = lax.axis_index("s")

    def pass_dir(pw, base, idx2, htab, xtab, h_out, x_out):
        pltpu.sync_copy(idx2.at[pl.ds(base, pw)], idx_v.at[pl.ds(0, pw)])

        def fire_g(s, hb, xb, sem):
            pltpu.async_copy(htab.at[idx_v.at[s]], hb, sem)
            pltpu.async_copy(xtab.at[idx_v.at[s]], xb, sem)

        def wait_g(hb, xb, sem):
            pltpu.make_async_copy(htab.at[idx_v.at[0]], hb, sem).wait()
            pltpu.make_async_copy(xtab.at[idx_v.at[0]], xb, sem).wait()

        def fire_w(s, hb, xb, sem):
            pltpu.async_copy(hb, h_out.at[pl.ds((base + s) * CB, CB)], sem)
            pltpu.async_copy(xb, x_out.at[pl.ds((base + s) * CB, CB)], sem)

        def wait_w(hb, xb, sem):
            pltpu.make_async_copy(hb, h_out.at[pl.ds(0, CB)], sem).wait()
            pltpu.make_async_copy(xb, x_out.at[pl.ds(0, CB)], sem).wait()

        fire_g(0, hb_a, xb_a, gsa)
        fire_g(1, hb_b, xb_b, gsb)

        def body(g, carry):
            wait_g(hb_a, xb_a, gsa)
            fire_w(2 * g, hb_a, xb_a, wsa)
            wait_g(hb_b, xb_b, gsb)
            fire_w(2 * g + 1, hb_b, xb_b, wsb)

            @pl.when(g < pw // 2 - 1)
            def _():
                wait_w(hb_a, xb_a, wsa)
                fire_g(2 * g + 2, hb_a, xb_a, gsa)
                wait_w(hb_b, xb_b, wsb)
                fire_g(2 * g + 3, hb_b, xb_b, gsb)

            return carry

        lax.fori_loop(0, pw // 2, body, 0)
        wait_w(hb_a, xb_a, wsa)
        wait_w(hb_b, xb_b, wsb)

    def all_passes(ll_pw, ll_base, rl_pw, rl_base):
        pass_dir(ll_pw, ll_base, src_ll, h_lig, xl, hs_ll, xs_ll)
        pass_dir(ll_pw, ll_base, dst_ll, h_lig, xl, hd_ll, xd_ll)
        pass_dir(rl_pw, rl_base, src_rl, h_rec, xr, hs_rl, xs_rl)
        pass_dir(rl_pw, rl_base, dst_rl, h_lig, xl, hd_rl, xd_rl)

    @pl.when(cid == 0)
    def _():
        all_passes(LL0, sid * LL0, RL0, sid * RL0)

    @pl.when(cid == 1)
    def _():
        all_passes(LL1, NS * LL0 + sid * LL1, RL1, NS * RL0 + sid * RL1)


_EB = 512


def _edge_mlp_body(hs_ref, hd_ref, xs_ref, xd_ref,
                   w1a, w1b, w1r, b1, w2, b2,
                   v1a, v1b, v1r, c1, w2c, c2,
                   mh_ref, mx_ref):
    hs = hs_ref[...]
    hd = hd_ref[...]
    diff = xs_ref[...] - xd_ref[...]
    d2 = jnp.sum(diff * diff, axis=1, keepdims=True)
    dij = jnp.sqrt(d2)
    xn = diff / (dij + 1e-9)
    pre = (jnp.dot(hs, w1a[...], preferred_element_type=jnp.float32)
           + jnp.dot(hd, w1b[...], preferred_element_type=jnp.float32)
           + dij * w1r[...] + b1[...])
    e1 = _silu(pre)
    mh = _silu(jnp.dot(e1, w2[...], preferred_element_type=jnp.float32) + b2[...])
    prec = (jnp.dot(hs, v1a[...], preferred_element_type=jnp.float32)
            + jnp.dot(hd, v1b[...], preferred_element_type=jnp.float32)
            + dij * v1r[...] + c1[...])
    e1c = _silu(prec)
    cc = _silu(jnp.dot(e1c, w2c[...], preferred_element_type=jnp.float32) + c2[...])
    mh_ref[...] = mh
    mx_ref[...] = cc[:, 0:1] * xn


def _edge_mlp(ep, hs, hd, xs, xd, ws):
    eb = lambda i: (i, 0)
    wb = lambda i: (0, 0)
    return pl.pallas_call(
        _edge_mlp_body,
        grid=(ep // _EB,),
        in_specs=[
            pl.BlockSpec((_EB, D), eb), pl.BlockSpec((_EB, D), eb),
            pl.BlockSpec((_EB, XW), eb), pl.BlockSpec((_EB, XW), eb),
            pl.BlockSpec((D, H), wb), pl.BlockSpec((D, H), wb),
            pl.BlockSpec((1, H), wb), pl.BlockSpec((1, H), wb),
            pl.BlockSpec((H, H), wb), pl.BlockSpec((1, H), wb),
            pl.BlockSpec((D, H), wb), pl.BlockSpec((D, H), wb),
            pl.BlockSpec((1, H), wb), pl.BlockSpec((1, H), wb),
            pl.BlockSpec((H, XW), wb), pl.BlockSpec((1, XW), wb),
        ],
        out_specs=[
            pl.BlockSpec((_EB, H), eb),
            pl.BlockSpec((_EB, XW), eb),
        ],
        out_shape=[
            jax.ShapeDtypeStruct((ep, H), jnp.float32),
            jax.ShapeDtypeStruct((ep, XW), jnp.float32),
        ],
    )(hs, hd, xs, xd, *ws)


@functools.partial(
    pl.kernel,
    out_type=(
        jax.ShapeDtypeStruct((NC, ACC_ROWS, D), jnp.float32),
        jax.ShapeDtypeStruct((NC, ACC_ROWS, XW), jnp.float32),
    ),
    mesh=_sc_mesh,
    compiler_params=_sc_params,
    scratch_types=[
        pltpu.VMEM((1, CB), jnp.int32),
        pltpu.VMEM((1, CB), jnp.int32),
        pltpu.VMEM((CB, D), jnp.float32),
        pltpu.VMEM((CB, D), jnp.float32),
        pltpu.VMEM((CB, XW), jnp.float32),
        pltpu.VMEM((CB, XW), jnp.float32),
        pltpu.VMEM_SHARED((ACC_ROWS, D), jnp.float32),
        pltpu.VMEM_SHARED((ACC_ROWS, XW), jnp.float32),
        pltpu.SemaphoreType.DMA,
        pltpu.SemaphoreType.DMA,
        pltpu.SemaphoreType.DMA,
        pltpu.SemaphoreType.DMA,
    ],
)
def _scatter_kernel(mh_ll, mx_ll, mh_rl, mx_rl, dsts_ll, dsts_rl, zh, zx,
                    part_h, part_x, idx_a, idx_b, hb_a, hb_b, xb_a, xb_b,
                    acc_h, acc_x, rsa, rsb, ssa, ssb):
    cid = lax.axis_index("c")
    sid = lax.axis_index("s")
    wid = sid * NC + cid
    r0 = sid * RPS
    pltpu.sync_copy(zh.at[pl.ds(r0, RPS)], acc_h.at[pl.ds(r0, RPS)])
    pltpu.sync_copy(zx.at[pl.ds(r0, RPS)], acc_x.at[pl.ds(r0, RPS)])
    plsc.subcore_barrier()

    def run(pw, dst3, mh_hbm, mx_hbm):
        base = wid * pw

        def fire_r(s, hb, xb, idxb, sem):
            pltpu.async_copy(mh_hbm.at[pl.ds((base + s) * CB, CB)], hb, sem)
            pltpu.async_copy(mx_hbm.at[pl.ds((base + s) * CB, CB)], xb, sem)
            pltpu.async_copy(dst3.at[wid, pl.ds(s, 1)], idxb, sem)

        def wait_r(hb, xb, idxb, sem):
            pltpu.make_async_copy(mh_hbm.at[pl.ds(0, CB)], hb, sem).wait()
            pltpu.make_async_copy(mx_hbm.at[pl.ds(0, CB)], xb, sem).wait()
            pltpu.make_async_copy(dst3.at[0, pl.ds(0, 1)], idxb, sem).wait()

        def do_sadd(hb, xb, idxb, sem):
            pltpu.async_copy(hb, acc_h.at[idxb.at[0]], sem, add=True)
            pltpu.async_copy(xb, acc_x.at[idxb.at[0]], sem, add=True)
            pltpu.make_async_copy(hb, acc_h.at[idxb.at[0]], sem).wait()
            pltpu.make_async_copy(xb, acc_x.at[idxb.at[0]], sem).wait()

        fire_r(0, hb_a, xb_a, idx_a, rsa)
        fire_r(1, hb_b, xb_b, idx_b, rsb)

        def body(g, carry):
            wait_r(hb_a, xb_a, idx_a, rsa)
            do_sadd(hb_a, xb_a, idx_a, ssa)

            @pl.when(g < pw // 2 - 1)
            def _():
                fire_r(2 * g + 2, hb_a, xb_a, idx_a, rsa)

            wait_r(hb_b, xb_b, idx_b, rsb)
            do_sadd(hb_b, xb_b, idx_b, ssb)

            @pl.when(g < pw // 2 - 1)
            def _():
                fire_r(2 * g + 3, hb_b, xb_b, idx_b, rsb)

            return carry

        lax.fori_loop(0, pw // 2, body, 0)

    run(PW_LL, dsts_ll, mh_ll, mx_ll)
    run(PW_RL, dsts_rl, mh_rl, mx_rl)
    plsc.subcore_barrier()
    pltpu.sync_copy(acc_h.at[pl.ds(r0, RPS)], part_h.at[cid, pl.ds(r0, RPS)])
    pltpu.sync_copy(acc_x.at[pl.ds(r0, RPS)], part_x.at[cid, pl.ds(r0, RPS)])


_NB = 1000


def _node_body(h_ref, ph0, ph1, xl_ref, px0, px1,
               wn1a, wn1b, bn1, wn2, bn2, ho_ref, xo_ref):
    h = h_ref[...]
    hn = ph0[...] + ph1[...]
    pre = (jnp.dot(h, wn1a[...], preferred_element_type=jnp.float32)
           + jnp.dot(hn, wn1b[...], preferred_element_type=jnp.float32)
           + bn1[...])
    m = jnp.dot(_silu(pre), wn2[...], preferred_element_type=jnp.float32) + bn2[...]
    ho_ref[...] = h + m
    xo_ref[...] = xl_ref[...] + px0[...] + px1[...]


def _node_call(h_lig, ph0, ph1, xl, px0, px1, wn1a, wn1b, bn1, wn2, bn2):
    nb = lambda i: (i, 0)
    wb = lambda i: (0, 0)
    return pl.pallas_call(
        _node_body,
        grid=(N_LIG // _NB,),
        in_specs=[
            pl.BlockSpec((_NB, D), nb), pl.BlockSpec((_NB, D), nb),
            pl.BlockSpec((_NB, D), nb),
            pl.BlockSpec((_NB, XW), nb), pl.BlockSpec((_NB, XW), nb),
            pl.BlockSpec((_NB, XW), nb),
            pl.BlockSpec((D, H), wb), pl.BlockSpec((D, H), wb),
            pl.BlockSpec((1, H), wb), pl.BlockSpec((H, D), wb),
            pl.BlockSpec((1, D), wb),
        ],
        out_specs=[
            pl.BlockSpec((_NB, D), nb),
            pl.BlockSpec((_NB, XW), nb),
        ],
        out_shape=[
            jax.ShapeDtypeStruct((N_LIG, D), jnp.float32),
            jax.ShapeDtypeStruct((N_LIG, XW), jnp.float32),
        ],
    )(h_lig, ph0, ph1, xl, px0, px1, wn1a, wn1b, bn1, wn2, bn2)


def _prep_idx(ei, ep):
    e = ei.shape[1]
    src = jnp.pad(ei[0], (0, ep - e))
    dst_g = jnp.pad(ei[1], (0, ep - e))
    dst_s = jnp.pad(ei[1], (0, ep - e), constant_values=N_LIG)
    return (src.reshape(-1, CB), dst_g.reshape(-1, CB),
            dst_s.reshape(NW, -1, CB))


def kernel(h_lig, h_rec, x_lig, x_rec, edge_index_ll, edge_index_rl,
           W1e_ll, b1e_ll, W2e_ll, b2e_ll, W1c_ll, b1c_ll, W2c_ll, b2c_ll,
           W1e_rl, b1e_rl, W2e_rl, b2e_rl, W1c_rl, b1c_rl, W2c_rl, b2c_rl,
           Wn1, bn1, Wn2, bn2):
    xl = jnp.pad(x_lig, ((0, 0), (0, XW - 3)))
    xr = jnp.pad(x_rec, ((0, 0), (0, XW - 3)))
    src_ll, dstg_ll, dsts_ll = _prep_idx(edge_index_ll, EP_LL)
    src_rl, dstg_rl, dsts_rl = _prep_idx(edge_index_rl, EP_RL)

    (hs_ll, hd_ll, xs_ll, xd_ll,
     hs_rl, hd_rl, xs_rl, xd_rl) = _gather_kernel(
        h_lig, h_rec, xl, xr, src_ll, dstg_ll, src_rl, dstg_rl)

    def ws(W1e, b1e, W2e, b2e, W1c, b1c, W2c, b2c):
        return (W1e[:D], W1e[D:2 * D], W1e[2 * D:], b1e.reshape(1, H),
                W2e, b2e.reshape(1, H),
                W1c[:D], W1c[D:2 * D], W1c[2 * D:], b1c.reshape(1, H),
                jnp.pad(W2c, ((0, 0), (0, XW - 1))),
                jnp.pad(b2c, (0, XW - 1)).reshape(1, XW))

    mh_ll, mx_ll = _edge_mlp(EP_LL, hs_ll, hd_ll, xs_ll, xd_ll,
                             ws(W1e_ll, b1e_ll, W2e_ll, b2e_ll,
                                W1c_ll, b1c_ll, W2c_ll, b2c_ll))
    mh_rl, mx_rl = _edge_mlp(EP_RL, hs_rl, hd_rl, xs_rl, xd_rl,
                             ws(W1e_rl, b1e_rl, W2e_rl, b2e_rl,
                                W1c_rl, b1c_rl, W2c_rl, b2c_rl))

    zh = jnp.zeros((ACC_ROWS, D), jnp.float32)
    zx = jnp.zeros((ACC_ROWS, XW), jnp.float32)
    part_h, part_x = _scatter_kernel(mh_ll, mx_ll, mh_rl, mx_rl,
                                     dsts_ll, dsts_rl, zh, zx)

    h_out, xo = _node_call(h_lig, part_h[0], part_h[1], xl,
                           part_x[0], part_x[1],
                           Wn1[:D], Wn1[D:], bn1.reshape(1, H),
                           Wn2, bn2.reshape(1, D))
    return (h_out, h_rec, xo[:, :3], x_rec)


# speedup vs baseline: 1.2443x; 1.1344x over previous
import functools

import jax
import jax.numpy as jnp
from jax import lax
from jax.experimental import pallas as pl
from jax.experimental.pallas import tpu as pltpu
from jax.experimental.pallas import tpu_sc as plsc

N_LIG, N_REC, D, H = 10000, 40000, 128, 128
NC, NS = 2, 16
NW = NC * NS
CB = 128
XW = 16
PW_LL = 40
PW_RL = 100
EP_LL = NW * PW_LL * CB
EP_RL = NW * PW_RL * CB
LL0, LL1 = 60, 20
RL0, RL1 = 150, 50
PW_MAX = 150
RPS = 632
ACC_ROWS = RPS * NS

_sc_mesh = plsc.VectorSubcoreMesh(core_axis_name="c", subcore_axis_name="s")
_sc_params = pltpu.CompilerParams(use_tc_tiling_on_sc=False)


def _silu(x):
    return x / (1.0 + jnp.exp(-x))


@functools.partial(
    pl.kernel,
    out_type=(
        jax.ShapeDtypeStruct((EP_LL, D), jnp.float32),
        jax.ShapeDtypeStruct((EP_LL, D), jnp.float32),
        jax.ShapeDtypeStruct((EP_LL, XW), jnp.float32),
        jax.ShapeDtypeStruct((EP_LL, XW), jnp.float32),
        jax.ShapeDtypeStruct((EP_RL, D), jnp.float32),
        jax.ShapeDtypeStruct((EP_RL, D), jnp.float32),
        jax.ShapeDtypeStruct((EP_RL, XW), jnp.float32),
        jax.ShapeDtypeStruct((EP_RL, XW), jnp.float32),
    ),
    mesh=_sc_mesh,
    compiler_params=_sc_params,
    scratch_types=[
        pltpu.VMEM((PW_MAX, CB), jnp.int32),
        pltpu.VMEM((CB, D), jnp.float32),
        pltpu.VMEM((CB, D), jnp.float32),
        pltpu.VMEM((CB, XW), jnp.float32),
        pltpu.VMEM((CB, XW), jnp.float32),
        pltpu.SemaphoreType.DMA,
        pltpu.SemaphoreType.DMA,
        pltpu.SemaphoreType.DMA,
        pltpu.SemaphoreType.DMA,
    ],
)
def _gather_kernel(h_lig, h_rec, xl, xr,
                   src_ll, dst_ll, src_rl, dst_rl,
                   hs_ll, hd_ll, xs_ll, xd_ll,
                   hs_rl, hd_rl, xs_rl, xd_rl,
                   idx_v, hb_a, hb_b, xb_a, xb_b, gsa, gsb, wsa, wsb):
    cid = lax.axis_index("c")
    sid = lax.axis_index("s")

    def pass_dir(pw, base, idx2, htab, xtab, h_out, x_out):
        pltpu.sync_copy(idx2.at[pl.ds(base, pw)], idx_v.at[pl.ds(0, pw)])

        def fire_g(s, hb, xb, sem):
            pltpu.async_copy(htab.at[idx_v.at[s]], hb, sem)
            pltpu.async_copy(xtab.at[idx_v.at[s]], xb, sem)

        def wait_g(hb, xb, sem):
            pltpu.make_async_copy(htab.at[idx_v.at[0]], hb, sem).wait()
            pltpu.make_async_copy(xtab.at[idx_v.at[0]], xb, sem).wait()

        def fire_w(s, hb, xb, sem):
            pltpu.async_copy(hb, h_out.at[pl.ds((base + s) * CB, CB)], sem)
            pltpu.async_copy(xb, x_out.at[pl.ds((base + s) * CB, CB)], sem)

        def wait_w(hb, xb, sem):
            pltpu.make_async_copy(hb, h_out.at[pl.ds(0, CB)], sem).wait()
            pltpu.make_async_copy(xb, x_out.at[pl.ds(0, CB)], sem).wait()

        fire_g(0, hb_a, xb_a, gsa)
        fire_g(1, hb_b, xb_b, gsb)

        def body(g, carry):
            wait_g(hb_a, xb_a, gsa)
            fire_w(2 * g, hb_a, xb_a, wsa)
            wait_g(hb_b, xb_b, gsb)
            fire_w(2 * g + 1, hb_b, xb_b, wsb)

            @pl.when(g < pw // 2 - 1)
            def _():
                wait_w(hb_a, xb_a, wsa)
                fire_g(2 * g + 2, hb_a, xb_a, gsa)
                wait_w(hb_b, xb_b, wsb)
                fire_g(2 * g + 3, hb_b, xb_b, gsb)

            return carry

        lax.fori_loop(0, pw // 2, body, 0)
        wait_w(hb_a, xb_a, wsa)
        wait_w(hb_b, xb_b, wsb)

    def all_passes(ll_pw, ll_base, rl_pw, rl_base):
        pass_dir(ll_pw, ll_base, src_ll, h_lig, xl, hs_ll, xs_ll)
        pass_dir(ll_pw, ll_base, dst_ll, h_lig, xl, hd_ll, xd_ll)
        pass_dir(rl_pw, rl_base, src_rl, h_rec, xr, hs_rl, xs_rl)
        pass_dir(rl_pw, rl_base, dst_rl, h_lig, xl, hd_rl, xd_rl)

    @pl.when(cid == 0)
    def _():
        all_passes(LL0, sid * LL0, RL0, sid * RL0)

    @pl.when(cid == 1)
    def _():
        all_passes(LL1, NS * LL0 + sid * LL1, RL1, NS * RL0 + sid * RL1)


_EB = 512


_BG = _EB // 8


def _edge_mlp_body(hs_ref, hd_ref, xsg_ref, xdg_ref,
                   w1a, w1b, w1r, b1, w2, b2,
                   v1a, v1b, v1r, c1, w2c, c2,
                   mh_ref, mxg_ref):
    f32 = jnp.float32
    bf16 = jnp.bfloat16
    hs = hs_ref[...].astype(bf16)
    hd = hd_ref[...].astype(bf16)
    diffg = xsg_ref[...] - xdg_ref[...]
    i0 = lax.broadcasted_iota(jnp.int32, (D, D), 0)
    i1 = lax.broadcasted_iota(jnp.int32, (D, D), 1)
    sbd = (i0 // XW == i1 // XW).astype(f32)
    d2g = jnp.dot(diffg * diffg, sbd, preferred_element_type=f32)
    dijg = jnp.sqrt(d2g)
    xng = diffg / (dijg + 1e-9)
    e0 = lax.broadcasted_iota(jnp.int32, (_EB, _BG), 0)
    e1i = lax.broadcasted_iota(jnp.int32, (_EB, _BG), 1)
    lx = (e0 // 8 == e1i).astype(f32)
    t1 = jnp.dot(lx, dijg, preferred_element_type=f32)
    l0 = lax.broadcasted_iota(jnp.int32, (_EB, D), 0)
    l1 = lax.broadcasted_iota(jnp.int32, (_EB, D), 1)
    msk = (l1 == (l0 % 8) * XW).astype(f32)
    dij = jnp.sum(t1 * msk, axis=1, keepdims=True)
    pre = (jnp.dot(hs, w1a[...].astype(bf16), preferred_element_type=f32)
           + jnp.dot(hd, w1b[...].astype(bf16), preferred_element_type=f32)
           + dij * w1r[...] + b1[...])
    e1 = _silu(pre).astype(bf16)
    mh = _silu(jnp.dot(e1, w2[...].astype(bf16), preferred_element_type=f32)
               + b2[...])
    prec = (jnp.dot(hs, v1a[...].astype(bf16), preferred_element_type=f32)
            + jnp.dot(hd, v1b[...].astype(bf16), preferred_element_type=f32)
            + dij * v1r[...] + c1[...])
    e1c = _silu(prec).astype(bf16)
    cc = _silu(jnp.dot(e1c, w2c[...].astype(bf16), preferred_element_type=f32)
               + c2[...])
    c = cc[:, 0:1]
    bb = (l1 // XW == l0 % 8).astype(f32)
    lxt = (lax.broadcasted_iota(jnp.int32, (_BG, _EB), 1) // 8
           == lax.broadcasted_iota(jnp.int32, (_BG, _EB), 0)).astype(f32)
    cg = jnp.dot(lxt, c * bb, preferred_element_type=f32)
    mh_ref[...] = mh
    mxg_ref[...] = cg * xng


def _edge_mlp(ep, hs, hd, xsg, xdg, ws):
    eb = lambda i: (i, 0)
    wb = lambda i: (0, 0)
    return pl.pallas_call(
        _edge_mlp_body,
        grid=(ep // _EB,),
        in_specs=[
            pl.BlockSpec((_EB, D), eb), pl.BlockSpec((_EB, D), eb),
            pl.BlockSpec((_BG, D), eb), pl.BlockSpec((_BG, D), eb),
            pl.BlockSpec((D, H), wb), pl.BlockSpec((D, H), wb),
            pl.BlockSpec((1, H), wb), pl.BlockSpec((1, H), wb),
            pl.BlockSpec((H, H), wb), pl.BlockSpec((1, H), wb),
            pl.BlockSpec((D, H), wb), pl.BlockSpec((D, H), wb),
            pl.BlockSpec((1, H), wb), pl.BlockSpec((1, H), wb),
            pl.BlockSpec((H, XW), wb), pl.BlockSpec((1, XW), wb),
        ],
        out_specs=[
            pl.BlockSpec((_EB, H), eb),
            pl.BlockSpec((_BG, D), eb),
        ],
        out_shape=[
            jax.ShapeDtypeStruct((ep, H), jnp.float32),
            jax.ShapeDtypeStruct((ep // 8, D), jnp.float32),
        ],
    )(hs, hd, xsg, xdg, *ws)


@functools.partial(
    pl.kernel,
    out_type=(
        jax.ShapeDtypeStruct((NC, ACC_ROWS, D), jnp.float32),
        jax.ShapeDtypeStruct((NC, ACC_ROWS, XW), jnp.float32),
    ),
    mesh=_sc_mesh,
    compiler_params=_sc_params,
    scratch_types=[
        pltpu.VMEM((1, CB), jnp.int32),
        pltpu.VMEM((1, CB), jnp.int32),
        pltpu.VMEM((CB, D), jnp.float32),
        pltpu.VMEM((CB, D), jnp.float32),
        pltpu.VMEM((CB, XW), jnp.float32),
        pltpu.VMEM((CB, XW), jnp.float32),
        pltpu.VMEM_SHARED((ACC_ROWS, D), jnp.float32),
        pltpu.VMEM_SHARED((ACC_ROWS, XW), jnp.float32),
        pltpu.SemaphoreType.DMA,
        pltpu.SemaphoreType.DMA,
        pltpu.SemaphoreType.DMA,
        pltpu.SemaphoreType.DMA,
    ],
)
def _scatter_kernel(mh_ll, mx_ll, mh_rl, mx_rl, dsts_ll, dsts_rl, zh, zx,
                    part_h, part_x, idx_a, idx_b, hb_a, hb_b, xb_a, xb_b,
                    acc_h, acc_x, rsa, rsb, ssa, ssb):
    cid = lax.axis_index("c")
    sid = lax.axis_index("s")
    wid = sid * NC + cid
    r0 = sid * RPS
    pltpu.sync_copy(zh.at[pl.ds(r0, RPS)], acc_h.at[pl.ds(r0, RPS)])
    pltpu.sync_copy(zx.at[pl.ds(r0, RPS)], acc_x.at[pl.ds(r0, RPS)])
    plsc.subcore_barrier()

    def run(pw, dst3, mh_hbm, mx_hbm):
        base = wid * pw

        def fire_r(s, hb, xb, idxb, sem):
            pltpu.async_copy(mh_hbm.at[pl.ds((base + s) * CB, CB)], hb, sem)
            pltpu.async_copy(mx_hbm.at[pl.ds((base + s) * CB, CB)], xb, sem)
            pltpu.async_copy(dst3.at[wid, pl.ds(s, 1)], idxb, sem)

        def wait_r(hb, xb, idxb, sem):
            pltpu.make_async_copy(mh_hbm.at[pl.ds(0, CB)], hb, sem).wait()
            pltpu.make_async_copy(mx_hbm.at[pl.ds(0, CB)], xb, sem).wait()
            pltpu.make_async_copy(dst3.at[0, pl.ds(0, 1)], idxb, sem).wait()

        def do_sadd(hb, xb, idxb, sem):
            pltpu.async_copy(hb, acc_h.at[idxb.at[0]], sem, add=True)
            pltpu.async_copy(xb, acc_x.at[idxb.at[0]], sem, add=True)
            pltpu.make_async_copy(hb, acc_h.at[idxb.at[0]], sem).wait()
            pltpu.make_async_copy(xb, acc_x.at[idxb.at[0]], sem).wait()

        fire_r(0, hb_a, xb_a, idx_a, rsa)
        fire_r(1, hb_b, xb_b, idx_b, rsb)

        def body(g, carry):
            wait_r(hb_a, xb_a, idx_a, rsa)
            do_sadd(hb_a, xb_a, idx_a, ssa)

            @pl.when(g < pw // 2 - 1)
            def _():
                fire_r(2 * g + 2, hb_a, xb_a, idx_a, rsa)

            wait_r(hb_b, xb_b, idx_b, rsb)
            do_sadd(hb_b, xb_b, idx_b, ssb)

            @pl.when(g < pw // 2 - 1)
            def _():
                fire_r(2 * g + 3, hb_b, xb_b, idx_b, rsb)

            return carry

        lax.fori_loop(0, pw // 2, body, 0)

    run(PW_LL, dsts_ll, mh_ll, mx_ll)
    run(PW_RL, dsts_rl, mh_rl, mx_rl)
    plsc.subcore_barrier()
    pltpu.sync_copy(acc_h.at[pl.ds(r0, RPS)], part_h.at[cid, pl.ds(r0, RPS)])
    pltpu.sync_copy(acc_x.at[pl.ds(r0, RPS)], part_x.at[cid, pl.ds(r0, RPS)])


_NB = 1000


def _node_body(h_ref, ph0, ph1, xl_ref, px0, px1,
               wn1a, wn1b, bn1, wn2, bn2, ho_ref, xo_ref):
    h = h_ref[...]
    hn = ph0[...] + ph1[...]
    pre = (jnp.dot(h, wn1a[...], preferred_element_type=jnp.float32)
           + jnp.dot(hn, wn1b[...], preferred_element_type=jnp.float32)
           + bn1[...])
    m = jnp.dot(_silu(pre), wn2[...], preferred_element_type=jnp.float32) + bn2[...]
    ho_ref[...] = h + m
    xo_ref[...] = xl_ref[...] + px0[...] + px1[...]


def _node_call(h_lig, ph0, ph1, xl, px0, px1, wn1a, wn1b, bn1, wn2, bn2):
    nb = lambda i: (i, 0)
    wb = lambda i: (0, 0)
    return pl.pallas_call(
        _node_body,
        grid=(N_LIG // _NB,),
        in_specs=[
            pl.BlockSpec((_NB, D), nb), pl.BlockSpec((_NB, D), nb),
            pl.BlockSpec((_NB, D), nb),
            pl.BlockSpec((_NB, XW), nb), pl.BlockSpec((_NB, XW), nb),
            pl.BlockSpec((_NB, XW), nb),
            pl.BlockSpec((D, H), wb), pl.BlockSpec((D, H), wb),
            pl.BlockSpec((1, H), wb), pl.BlockSpec((H, D), wb),
            pl.BlockSpec((1, D), wb),
        ],
        out_specs=[
            pl.BlockSpec((_NB, D), nb),
            pl.BlockSpec((_NB, XW), nb),
        ],
        out_shape=[
            jax.ShapeDtypeStruct((N_LIG, D), jnp.float32),
            jax.ShapeDtypeStruct((N_LIG, XW), jnp.float32),
        ],
    )(h_lig, ph0, ph1, xl, px0, px1, wn1a, wn1b, bn1, wn2, bn2)


def _prep_idx(ei, ep):
    e = ei.shape[1]
    src = jnp.pad(ei[0], (0, ep - e))
    dst_g = jnp.pad(ei[1], (0, ep - e))
    dst_s = jnp.pad(ei[1], (0, ep - e), constant_values=N_LIG)
    return (src.reshape(-1, CB), dst_g.reshape(-1, CB),
            dst_s.reshape(NW, -1, CB))


def kernel(h_lig, h_rec, x_lig, x_rec, edge_index_ll, edge_index_rl,
           W1e_ll, b1e_ll, W2e_ll, b2e_ll, W1c_ll, b1c_ll, W2c_ll, b2c_ll,
           W1e_rl, b1e_rl, W2e_rl, b2e_rl, W1c_rl, b1c_rl, W2c_rl, b2c_rl,
           Wn1, bn1, Wn2, bn2):
    xl = jnp.pad(x_lig, ((0, 0), (0, XW - 3)))
    xr = jnp.pad(x_rec, ((0, 0), (0, XW - 3)))
    src_ll, dstg_ll, dsts_ll = _prep_idx(edge_index_ll, EP_LL)
    src_rl, dstg_rl, dsts_rl = _prep_idx(edge_index_rl, EP_RL)

    (hs_ll, hd_ll, xs_ll, xd_ll,
     hs_rl, hd_rl, xs_rl, xd_rl) = _gather_kernel(
        h_lig, h_rec, xl, xr, src_ll, dstg_ll, src_rl, dstg_rl)

    def ws(W1e, b1e, W2e, b2e, W1c, b1c, W2c, b2c):
        return (W1e[:D], W1e[D:2 * D], W1e[2 * D:], b1e.reshape(1, H),
                W2e, b2e.reshape(1, H),
                W1c[:D], W1c[D:2 * D], W1c[2 * D:], b1c.reshape(1, H),
                jnp.pad(W2c, ((0, 0), (0, XW - 1))),
                jnp.pad(b2c, (0, XW - 1)).reshape(1, XW))

    mh_ll, mxg_ll = _edge_mlp(EP_LL, hs_ll, hd_ll,
                              xs_ll.reshape(-1, D), xd_ll.reshape(-1, D),
                              ws(W1e_ll, b1e_ll, W2e_ll, b2e_ll,
                                 W1c_ll, b1c_ll, W2c_ll, b2c_ll))
    mh_rl, mxg_rl = _edge_mlp(EP_RL, hs_rl, hd_rl,
                              xs_rl.reshape(-1, D), xd_rl.reshape(-1, D),
                              ws(W1e_rl, b1e_rl, W2e_rl, b2e_rl,
                                 W1c_rl, b1c_rl, W2c_rl, b2c_rl))
    mx_ll = mxg_ll.reshape(-1, XW)
    mx_rl = mxg_rl.reshape(-1, XW)

    zh = jnp.zeros((ACC_ROWS, D), jnp.float32)
    zx = jnp.zeros((ACC_ROWS, XW), jnp.float32)
    part_h, part_x = _scatter_kernel(mh_ll, mx_ll, mh_rl, mx_rl,
                                     dsts_ll, dsts_rl, zh, zx)

    h_out, xo = _node_call(h_lig, part_h[0], part_h[1], xl,
                           part_x[0], part_x[1],
                           Wn1[:D], Wn1[D:], bn1.reshape(1, H),
                           Wn2, bn2.reshape(1, D))
    return (h_out, h_rec, xo[:, :3], x_rec)


# speedup vs baseline: 1.4601x; 1.1735x over previous
import functools

import jax
import jax.numpy as jnp
from jax import lax
from jax.experimental import pallas as pl
from jax.experimental.pallas import tpu as pltpu
from jax.experimental.pallas import tpu_sc as plsc

N_LIG, N_REC, D, H = 10000, 40000, 128, 128
NC, NS = 2, 16
NW = NC * NS
CB = 128
XW = 16
PW_LL = 40
PW_RL = 100
EP_LL = NW * PW_LL * CB
EP_RL = NW * PW_RL * CB
LL0, LL1 = 60, 20
RL0, RL1 = 150, 50
PW_MAX = 150
RPS = 632
ACC_ROWS = RPS * NS

_sc_mesh = plsc.VectorSubcoreMesh(core_axis_name="c", subcore_axis_name="s")
_sc_params = pltpu.CompilerParams(use_tc_tiling_on_sc=False)


def _silu(x):
    return x / (1.0 + jnp.exp(-x))


@functools.partial(
    pl.kernel,
    out_type=(
        jax.ShapeDtypeStruct((EP_LL, D), jnp.float32),
        jax.ShapeDtypeStruct((EP_LL, D), jnp.float32),
        jax.ShapeDtypeStruct((EP_LL, XW), jnp.float32),
        jax.ShapeDtypeStruct((EP_LL, XW), jnp.float32),
        jax.ShapeDtypeStruct((EP_RL, D), jnp.float32),
        jax.ShapeDtypeStruct((EP_RL, D), jnp.float32),
        jax.ShapeDtypeStruct((EP_RL, XW), jnp.float32),
        jax.ShapeDtypeStruct((EP_RL, XW), jnp.float32),
    ),
    mesh=_sc_mesh,
    compiler_params=_sc_params,
    scratch_types=[
        pltpu.VMEM((PW_MAX, CB), jnp.int32),
        pltpu.VMEM((CB, D), jnp.float32),
        pltpu.VMEM((CB, D), jnp.float32),
        pltpu.VMEM((CB, XW), jnp.float32),
        pltpu.VMEM((CB, XW), jnp.float32),
        pltpu.SemaphoreType.DMA,
        pltpu.SemaphoreType.DMA,
        pltpu.SemaphoreType.DMA,
        pltpu.SemaphoreType.DMA,
    ],
)
def _gather_kernel(h_lig, h_rec, xl, xr,
                   src_ll, dst_ll, src_rl, dst_rl,
                   hs_ll, hd_ll, xs_ll, xd_ll,
                   hs_rl, hd_rl, xs_rl, xd_rl,
                   idx_v, hb_a, hb_b, xb_a, xb_b, gsa, gsb, wsa, wsb):
    cid = lax.axis_index("c")
    sid = lax.axis_index("s")

    def pass_dir(pw, base, idx2, htab, xtab, h_out, x_out):
        pltpu.sync_copy(idx2.at[pl.ds(base, pw)], idx_v.at[pl.ds(0, pw)])

        def fire_g(s, hb, xb, sem):
            pltpu.async_copy(htab.at[idx_v.at[s]], hb, sem)
            pltpu.async_copy(xtab.at[idx_v.at[s]], xb, sem)

        def wait_g(hb, xb, sem):
            pltpu.make_async_copy(htab.at[idx_v.at[0]], hb, sem).wait()
            pltpu.make_async_copy(xtab.at[idx_v.at[0]], xb, sem).wait()

        def fire_w(s, hb, xb, sem):
            pltpu.async_copy(hb, h_out.at[pl.ds((base + s) * CB, CB)], sem)
            pltpu.async_copy(xb, x_out.at[pl.ds((base + s) * CB, CB)], sem)

        def wait_w(hb, xb, sem):
            pltpu.make_async_copy(hb, h_out.at[pl.ds(0, CB)], sem).wait()
            pltpu.make_async_copy(xb, x_out.at[pl.ds(0, CB)], sem).wait()

        fire_g(0, hb_a, xb_a, gsa)
        fire_g(1, hb_b, xb_b, gsb)

        def body(g, carry):
            wait_g(hb_a, xb_a, gsa)
            fire_w(2 * g, hb_a, xb_a, wsa)
            wait_g(hb_b, xb_b, gsb)
            fire_w(2 * g + 1, hb_b, xb_b, wsb)

            @pl.when(g < pw // 2 - 1)
            def _():
                wait_w(hb_a, xb_a, wsa)
                fire_g(2 * g + 2, hb_a, xb_a, gsa)
                wait_w(hb_b, xb_b, wsb)
                fire_g(2 * g + 3, hb_b, xb_b, gsb)

            return carry

        lax.fori_loop(0, pw // 2, body, 0)
        wait_w(hb_a, xb_a, wsa)
        wait_w(hb_b, xb_b, wsb)

    def all_passes(ll_pw, ll_base, rl_pw, rl_base):
        pass_dir(ll_pw, ll_base, src_ll, h_lig, xl, hs_ll, xs_ll)
        pass_dir(ll_pw, ll_base, dst_ll, h_lig, xl, hd_ll, xd_ll)
        pass_dir(rl_pw, rl_base, src_rl, h_rec, xr, hs_rl, xs_rl)
        pass_dir(rl_pw, rl_base, dst_rl, h_lig, xl, hd_rl, xd_rl)

    @pl.when(cid == 0)
    def _():
        all_passes(LL0, sid * LL0, RL0, sid * RL0)

    @pl.when(cid == 1)
    def _():
        all_passes(LL1, NS * LL0 + sid * LL1, RL1, NS * RL0 + sid * RL1)


_EB = 2048


_BG = _EB // 8


def _edge_mlp_body(hs_ref, hd_ref, xsg_ref, xdg_ref,
                   w1a, w1b, w1r, b1, w2, b2,
                   v1a, v1b, v1r, c1, w2c, c2,
                   mh_ref, mxg_ref):
    f32 = jnp.float32
    bf16 = jnp.bfloat16
    hs = hs_ref[...].astype(bf16)
    hd = hd_ref[...].astype(bf16)
    diffg = xsg_ref[...] - xdg_ref[...]
    i0 = lax.broadcasted_iota(jnp.int32, (D, D), 0)
    i1 = lax.broadcasted_iota(jnp.int32, (D, D), 1)
    sbd = (i0 // XW == i1 // XW).astype(f32)
    d2g = jnp.dot(diffg * diffg, sbd, preferred_element_type=f32)
    dijg = jnp.sqrt(d2g)
    xng = diffg / (dijg + 1e-9)
    e0 = lax.broadcasted_iota(jnp.int32, (_EB, _BG), 0)
    e1i = lax.broadcasted_iota(jnp.int32, (_EB, _BG), 1)
    lx = (e0 // 8 == e1i).astype(f32)
    t1 = jnp.dot(lx, dijg, preferred_element_type=f32)
    l0 = lax.broadcasted_iota(jnp.int32, (_EB, D), 0)
    l1 = lax.broadcasted_iota(jnp.int32, (_EB, D), 1)
    msk = (l1 == (l0 % 8) * XW).astype(f32)
    dij = jnp.sum(t1 * msk, axis=1, keepdims=True)
    pre = (jnp.dot(hs, w1a[...].astype(bf16), preferred_element_type=f32)
           + jnp.dot(hd, w1b[...].astype(bf16), preferred_element_type=f32)
           + dij * w1r[...] + b1[...])
    e1 = _silu(pre).astype(bf16)
    mh = _silu(jnp.dot(e1, w2[...].astype(bf16), preferred_element_type=f32)
               + b2[...])
    prec = (jnp.dot(hs, v1a[...].astype(bf16), preferred_element_type=f32)
            + jnp.dot(hd, v1b[...].astype(bf16), preferred_element_type=f32)
            + dij * v1r[...] + c1[...])
    e1c = _silu(prec).astype(bf16)
    cc = _silu(jnp.dot(e1c, w2c[...].astype(bf16), preferred_element_type=f32)
               + c2[...])
    c = cc[:, 0:1]
    bb = (l1 // XW == l0 % 8).astype(f32)
    lxt = (lax.broadcasted_iota(jnp.int32, (_BG, _EB), 1) // 8
           == lax.broadcasted_iota(jnp.int32, (_BG, _EB), 0)).astype(f32)
    cg = jnp.dot(lxt, c * bb, preferred_element_type=f32)
    mh_ref[...] = mh
    mxg_ref[...] = cg * xng


def _edge_mlp(ep, hs, hd, xsg, xdg, ws):
    eb = lambda i: (i, 0)
    wb = lambda i: (0, 0)
    return pl.pallas_call(
        _edge_mlp_body,
        grid=(ep // _EB,),
        in_specs=[
            pl.BlockSpec((_EB, D), eb), pl.BlockSpec((_EB, D), eb),
            pl.BlockSpec((_BG, D), eb), pl.BlockSpec((_BG, D), eb),
            pl.BlockSpec((D, H), wb), pl.BlockSpec((D, H), wb),
            pl.BlockSpec((1, H), wb), pl.BlockSpec((1, H), wb),
            pl.BlockSpec((H, H), wb), pl.BlockSpec((1, H), wb),
            pl.BlockSpec((D, H), wb), pl.BlockSpec((D, H), wb),
            pl.BlockSpec((1, H), wb), pl.BlockSpec((1, H), wb),
            pl.BlockSpec((H, XW), wb), pl.BlockSpec((1, XW), wb),
        ],
        out_specs=[
            pl.BlockSpec((_EB, H), eb),
            pl.BlockSpec((_BG, D), eb),
        ],
        out_shape=[
            jax.ShapeDtypeStruct((ep, H), jnp.float32),
            jax.ShapeDtypeStruct((ep // 8, D), jnp.float32),
        ],
    )(hs, hd, xsg, xdg, *ws)


@functools.partial(
    pl.kernel,
    out_type=(
        jax.ShapeDtypeStruct((NC, ACC_ROWS, D), jnp.float32),
        jax.ShapeDtypeStruct((NC, ACC_ROWS, XW), jnp.float32),
    ),
    mesh=_sc_mesh,
    compiler_params=_sc_params,
    scratch_types=[
        pltpu.VMEM((1, CB), jnp.int32),
        pltpu.VMEM((1, CB), jnp.int32),
        pltpu.VMEM((CB, D), jnp.float32),
        pltpu.VMEM((CB, D), jnp.float32),
        pltpu.VMEM((CB, XW), jnp.float32),
        pltpu.VMEM((CB, XW), jnp.float32),
        pltpu.VMEM_SHARED((ACC_ROWS, D), jnp.float32),
        pltpu.VMEM_SHARED((ACC_ROWS, XW), jnp.float32),
        pltpu.SemaphoreType.DMA,
        pltpu.SemaphoreType.DMA,
        pltpu.SemaphoreType.DMA,
        pltpu.SemaphoreType.DMA,
    ],
)
def _scatter_kernel(mh_ll, mx_ll, mh_rl, mx_rl, dsts_ll, dsts_rl, zh, zx,
                    part_h, part_x, idx_a, idx_b, hb_a, hb_b, xb_a, xb_b,
                    acc_h, acc_x, rsa, rsb, ssa, ssb):
    cid = lax.axis_index("c")
    sid = lax.axis_index("s")
    wid = sid * NC + cid
    r0 = sid * RPS
    pltpu.sync_copy(zh.at[pl.ds(r0, RPS)], acc_h.at[pl.ds(r0, RPS)])
    pltpu.sync_copy(zx.at[pl.ds(r0, RPS)], acc_x.at[pl.ds(r0, RPS)])
    plsc.subcore_barrier()

    def run(pw, dst3, mh_hbm, mx_hbm):
        base = wid * pw

        def fire_r(s, hb, xb, idxb, sem):
            pltpu.async_copy(mh_hbm.at[pl.ds((base + s) * CB, CB)], hb, sem)
            pltpu.async_copy(mx_hbm.at[pl.ds((base + s) * CB, CB)], xb, sem)
            pltpu.async_copy(dst3.at[wid, pl.ds(s, 1)], idxb, sem)

        def wait_r(hb, xb, idxb, sem):
            pltpu.make_async_copy(mh_hbm.at[pl.ds(0, CB)], hb, sem).wait()
            pltpu.make_async_copy(mx_hbm.at[pl.ds(0, CB)], xb, sem).wait()
            pltpu.make_async_copy(dst3.at[0, pl.ds(0, 1)], idxb, sem).wait()

        def do_sadd(hb, xb, idxb, sem):
            pltpu.async_copy(hb, acc_h.at[idxb.at[0]], sem, add=True)
            pltpu.async_copy(xb, acc_x.at[idxb.at[0]], sem, add=True)
            pltpu.make_async_copy(hb, acc_h.at[idxb.at[0]], sem).wait()
            pltpu.make_async_copy(xb, acc_x.at[idxb.at[0]], sem).wait()

        fire_r(0, hb_a, xb_a, idx_a, rsa)
        fire_r(1, hb_b, xb_b, idx_b, rsb)

        def body(g, carry):
            wait_r(hb_a, xb_a, idx_a, rsa)
            do_sadd(hb_a, xb_a, idx_a, ssa)

            @pl.when(g < pw // 2 - 1)
            def _():
                fire_r(2 * g + 2, hb_a, xb_a, idx_a, rsa)

            wait_r(hb_b, xb_b, idx_b, rsb)
            do_sadd(hb_b, xb_b, idx_b, ssb)

            @pl.when(g < pw // 2 - 1)
            def _():
                fire_r(2 * g + 3, hb_b, xb_b, idx_b, rsb)

            return carry

        lax.fori_loop(0, pw // 2, body, 0)

    run(PW_LL, dsts_ll, mh_ll, mx_ll)
    run(PW_RL, dsts_rl, mh_rl, mx_rl)
    plsc.subcore_barrier()
    pltpu.sync_copy(acc_h.at[pl.ds(r0, RPS)], part_h.at[cid, pl.ds(r0, RPS)])
    pltpu.sync_copy(acc_x.at[pl.ds(r0, RPS)], part_x.at[cid, pl.ds(r0, RPS)])


_NB = 1000


def _node_body(h_ref, ph0, ph1, xl_ref, px0, px1,
               wn1a, wn1b, bn1, wn2, bn2, ho_ref, xo_ref):
    h = h_ref[...]
    hn = ph0[...] + ph1[...]
    pre = (jnp.dot(h, wn1a[...], preferred_element_type=jnp.float32)
           + jnp.dot(hn, wn1b[...], preferred_element_type=jnp.float32)
           + bn1[...])
    m = jnp.dot(_silu(pre), wn2[...], preferred_element_type=jnp.float32) + bn2[...]
    ho_ref[...] = h + m
    xo_ref[...] = xl_ref[...] + px0[...] + px1[...]


def _node_call(h_lig, ph0, ph1, xl, px0, px1, wn1a, wn1b, bn1, wn2, bn2):
    nb = lambda i: (i, 0)
    wb = lambda i: (0, 0)
    return pl.pallas_call(
        _node_body,
        grid=(N_LIG // _NB,),
        in_specs=[
            pl.BlockSpec((_NB, D), nb), pl.BlockSpec((_NB, D), nb),
            pl.BlockSpec((_NB, D), nb),
            pl.BlockSpec((_NB, XW), nb), pl.BlockSpec((_NB, XW), nb),
            pl.BlockSpec((_NB, XW), nb),
            pl.BlockSpec((D, H), wb), pl.BlockSpec((D, H), wb),
            pl.BlockSpec((1, H), wb), pl.BlockSpec((H, D), wb),
            pl.BlockSpec((1, D), wb),
        ],
        out_specs=[
            pl.BlockSpec((_NB, D), nb),
            pl.BlockSpec((_NB, XW), nb),
        ],
        out_shape=[
            jax.ShapeDtypeStruct((N_LIG, D), jnp.float32),
            jax.ShapeDtypeStruct((N_LIG, XW), jnp.float32),
        ],
    )(h_lig, ph0, ph1, xl, px0, px1, wn1a, wn1b, bn1, wn2, bn2)


def _prep_idx(ei, ep):
    e = ei.shape[1]
    src = jnp.pad(ei[0], (0, ep - e))
    dst_g = jnp.pad(ei[1], (0, ep - e))
    dst_s = jnp.pad(ei[1], (0, ep - e), constant_values=N_LIG)
    return (src.reshape(-1, CB), dst_g.reshape(-1, CB),
            dst_s.reshape(NW, -1, CB))


def kernel(h_lig, h_rec, x_lig, x_rec, edge_index_ll, edge_index_rl,
           W1e_ll, b1e_ll, W2e_ll, b2e_ll, W1c_ll, b1c_ll, W2c_ll, b2c_ll,
           W1e_rl, b1e_rl, W2e_rl, b2e_rl, W1c_rl, b1c_rl, W2c_rl, b2c_rl,
           Wn1, bn1, Wn2, bn2):
    xl = jnp.pad(x_lig, ((0, 0), (0, XW - 3)))
    xr = jnp.pad(x_rec, ((0, 0), (0, XW - 3)))
    src_ll, dstg_ll, dsts_ll = _prep_idx(edge_index_ll, EP_LL)
    src_rl, dstg_rl, dsts_rl = _prep_idx(edge_index_rl, EP_RL)

    (hs_ll, hd_ll, xs_ll, xd_ll,
     hs_rl, hd_rl, xs_rl, xd_rl) = _gather_kernel(
        h_lig, h_rec, xl, xr, src_ll, dstg_ll, src_rl, dstg_rl)

    def ws(W1e, b1e, W2e, b2e, W1c, b1c, W2c, b2c):
        return (W1e[:D], W1e[D:2 * D], W1e[2 * D:], b1e.reshape(1, H),
                W2e, b2e.reshape(1, H),
                W1c[:D], W1c[D:2 * D], W1c[2 * D:], b1c.reshape(1, H),
                jnp.pad(W2c, ((0, 0), (0, XW - 1))),
                jnp.pad(b2c, (0, XW - 1)).reshape(1, XW))

    mh_ll, mxg_ll = _edge_mlp(EP_LL, hs_ll, hd_ll,
                              xs_ll.reshape(-1, D), xd_ll.reshape(-1, D),
                              ws(W1e_ll, b1e_ll, W2e_ll, b2e_ll,
                                 W1c_ll, b1c_ll, W2c_ll, b2c_ll))
    mh_rl, mxg_rl = _edge_mlp(EP_RL, hs_rl, hd_rl,
                              xs_rl.reshape(-1, D), xd_rl.reshape(-1, D),
                              ws(W1e_rl, b1e_rl, W2e_rl, b2e_rl,
                                 W1c_rl, b1c_rl, W2c_rl, b2c_rl))
    mx_ll = mxg_ll.reshape(-1, XW)
    mx_rl = mxg_rl.reshape(-1, XW)

    zh = jnp.zeros((ACC_ROWS, D), jnp.float32)
    zx = jnp.zeros((ACC_ROWS, XW), jnp.float32)
    part_h, part_x = _scatter_kernel(mh_ll, mx_ll, mh_rl, mx_rl,
                                     dsts_ll, dsts_rl, zh, zx)

    h_out, xo = _node_call(h_lig, part_h[0], part_h[1], xl,
                           part_x[0], part_x[1],
                           Wn1[:D], Wn1[D:], bn1.reshape(1, H),
                           Wn2, bn2.reshape(1, D))
    return (h_out, h_rec, xo[:, :3], x_rec)


# speedup vs baseline: 2.1201x; 1.4520x over previous
import functools

import jax
import jax.numpy as jnp
from jax import lax
from jax.experimental import pallas as pl
from jax.experimental.pallas import tpu as pltpu
from jax.experimental.pallas import tpu_sc as plsc

N_LIG, N_REC, D, H = 10000, 40000, 128, 128
NC, NS = 2, 16
NW = NC * NS
CB = 128
XW = 16
PW_LL = 40
PW_RL = 100
EP_LL = NW * PW_LL * CB
EP_RL = NW * PW_RL * CB
LL0, LL1 = 60, 20
RL0, RL1 = 150, 50
PW_MAX = 150
RPS = 632
ACC_ROWS = RPS * NS

_sc_mesh = plsc.VectorSubcoreMesh(core_axis_name="c", subcore_axis_name="s")
_sc_params = pltpu.CompilerParams(use_tc_tiling_on_sc=False)


def _silu(x):
    return x / (1.0 + jnp.exp(-x))


@functools.partial(
    pl.kernel,
    out_type=(
        jax.ShapeDtypeStruct((EP_LL, D), jnp.float32),
        jax.ShapeDtypeStruct((EP_LL, D), jnp.float32),
        jax.ShapeDtypeStruct((EP_LL, XW), jnp.float32),
        jax.ShapeDtypeStruct((EP_LL, XW), jnp.float32),
        jax.ShapeDtypeStruct((EP_RL, D), jnp.float32),
        jax.ShapeDtypeStruct((EP_RL, D), jnp.float32),
        jax.ShapeDtypeStruct((EP_RL, XW), jnp.float32),
        jax.ShapeDtypeStruct((EP_RL, XW), jnp.float32),
    ),
    mesh=_sc_mesh,
    compiler_params=_sc_params,
    scratch_types=[
        pltpu.VMEM((1, CB), jnp.int32),
        pltpu.VMEM((1, CB), jnp.int32),
        pltpu.VMEM((CB, D), jnp.float32),
        pltpu.VMEM((CB, D), jnp.float32),
        pltpu.VMEM((CB, XW), jnp.float32),
        pltpu.VMEM((CB, XW), jnp.float32),
        pltpu.VMEM_SHARED((N_LIG, D), jnp.float32),
        pltpu.VMEM_SHARED((N_LIG, XW), jnp.float32),
        pltpu.SemaphoreType.DMA,
        pltpu.SemaphoreType.DMA,
        pltpu.SemaphoreType.DMA,
        pltpu.SemaphoreType.DMA,
        pltpu.SemaphoreType.DMA,
        pltpu.SemaphoreType.DMA,
    ],
)
def _gather_kernel(h_lig, h_rec, xl, xr,
                   src_ll, dst_ll, src_rl, dst_rl,
                   hs_ll, hd_ll, xs_ll, xd_ll,
                   hs_rl, hd_rl, xs_rl, xd_rl,
                   idx_a, idx_b, hb_a, hb_b, xb_a, xb_b, sp_h, sp_x,
                   isa, isb, gsa, gsb, wsa, wsb):
    cid = lax.axis_index("c")
    sid = lax.axis_index("s")
    wid = sid * NC + cid

    @pl.when(sid < NS - 1)
    def _():
        pltpu.sync_copy(h_lig.at[pl.ds(sid * 632, 632)],
                        sp_h.at[pl.ds(sid * 632, 632)])
        pltpu.sync_copy(xl.at[pl.ds(sid * 632, 632)],
                        sp_x.at[pl.ds(sid * 632, 632)])

    @pl.when(sid == NS - 1)
    def _():
        pltpu.sync_copy(h_lig.at[pl.ds(9480, 520)],
                        sp_h.at[pl.ds(9480, 520)])
        pltpu.sync_copy(xl.at[pl.ds(9480, 520)],
                        sp_x.at[pl.ds(9480, 520)])

    plsc.subcore_barrier()

    def pass_dir(pw, base, idx2, htab, xtab, h_out, x_out):
        def fire_i(s, idxb, sem):
            pltpu.async_copy(idx2.at[pl.ds(base + s, 1)], idxb, sem)

        def wait_i(idxb, sem):
            pltpu.make_async_copy(idx2.at[pl.ds(0, 1)], idxb, sem).wait()

        def fire_g(idxb, hb, xb, sem):
            pltpu.async_copy(htab.at[idxb.at[0]], hb, sem)
            pltpu.async_copy(xtab.at[idxb.at[0]], xb, sem)

        def wait_g(idxb, hb, xb, sem):
            pltpu.make_async_copy(htab.at[idxb.at[0]], hb, sem).wait()
            pltpu.make_async_copy(xtab.at[idxb.at[0]], xb, sem).wait()

        def fire_w(s, hb, xb, sem):
            pltpu.async_copy(hb, h_out.at[pl.ds((base + s) * CB, CB)], sem)
            pltpu.async_copy(xb, x_out.at[pl.ds((base + s) * CB, CB)], sem)

        def wait_w(hb, xb, sem):
            pltpu.make_async_copy(hb, h_out.at[pl.ds(0, CB)], sem).wait()
            pltpu.make_async_copy(xb, x_out.at[pl.ds(0, CB)], sem).wait()

        fire_i(0, idx_a, isa)
        fire_i(1, idx_b, isb)

        def body(g, carry):
            wait_i(idx_a, isa)

            @pl.when(g > 0)
            def _():
                wait_w(hb_a, xb_a, wsa)

            fire_g(idx_a, hb_a, xb_a, gsa)
            wait_i(idx_b, isb)

            @pl.when(g > 0)
            def _():
                wait_w(hb_b, xb_b, wsb)

            fire_g(idx_b, hb_b, xb_b, gsb)
            wait_g(idx_a, hb_a, xb_a, gsa)
            fire_w(2 * g, hb_a, xb_a, wsa)

            @pl.when(g < pw // 2 - 1)
            def _():
                fire_i(2 * g + 2, idx_a, isa)

            wait_g(idx_b, hb_b, xb_b, gsb)
            fire_w(2 * g + 1, hb_b, xb_b, wsb)

            @pl.when(g < pw // 2 - 1)
            def _():
                fire_i(2 * g + 3, idx_b, isb)

            return carry

        lax.fori_loop(0, pw // 2, body, 0)
        wait_w(hb_a, xb_a, wsa)
        wait_w(hb_b, xb_b, wsb)

    pass_dir(PW_LL, wid * PW_LL, src_ll, sp_h, sp_x, hs_ll, xs_ll)
    pass_dir(PW_LL, wid * PW_LL, dst_ll, sp_h, sp_x, hd_ll, xd_ll)
    pass_dir(PW_RL, wid * PW_RL, dst_rl, sp_h, sp_x, hd_rl, xd_rl)

    @pl.when(cid == 0)
    def _():
        pass_dir(RL0, sid * RL0, src_rl, h_rec, xr, hs_rl, xs_rl)

    @pl.when(cid == 1)
    def _():
        pass_dir(RL1, NS * RL0 + sid * RL1, src_rl, h_rec, xr, hs_rl, xs_rl)


_EB = 2048


_BG = _EB // 8


def _edge_mlp_body(hs_ref, hd_ref, xsg_ref, xdg_ref,
                   w1a, w1b, w1r, b1, w2, b2,
                   v1a, v1b, v1r, c1, w2c, c2,
                   mh_ref, mxg_ref):
    f32 = jnp.float32
    bf16 = jnp.bfloat16
    hs = hs_ref[...].astype(bf16)
    hd = hd_ref[...].astype(bf16)
    diffg = xsg_ref[...] - xdg_ref[...]
    i0 = lax.broadcasted_iota(jnp.int32, (D, D), 0)
    i1 = lax.broadcasted_iota(jnp.int32, (D, D), 1)
    sbd = (i0 // XW == i1 // XW).astype(f32)
    d2g = jnp.dot(diffg * diffg, sbd, preferred_element_type=f32)
    dijg = jnp.sqrt(d2g)
    xng = diffg / (dijg + 1e-9)
    e0 = lax.broadcasted_iota(jnp.int32, (_EB, _BG), 0)
    e1i = lax.broadcasted_iota(jnp.int32, (_EB, _BG), 1)
    lx = (e0 // 8 == e1i).astype(f32)
    t1 = jnp.dot(lx, dijg, preferred_element_type=f32)
    l0 = lax.broadcasted_iota(jnp.int32, (_EB, D), 0)
    l1 = lax.broadcasted_iota(jnp.int32, (_EB, D), 1)
    msk = (l1 == (l0 % 8) * XW).astype(f32)
    dij = jnp.sum(t1 * msk, axis=1, keepdims=True)
    pre = (jnp.dot(hs, w1a[...].astype(bf16), preferred_element_type=f32)
           + jnp.dot(hd, w1b[...].astype(bf16), preferred_element_type=f32)
           + dij * w1r[...] + b1[...])
    e1 = _silu(pre).astype(bf16)
    mh = _silu(jnp.dot(e1, w2[...].astype(bf16), preferred_element_type=f32)
               + b2[...])
    prec = (jnp.dot(hs, v1a[...].astype(bf16), preferred_element_type=f32)
            + jnp.dot(hd, v1b[...].astype(bf16), preferred_element_type=f32)
            + dij * v1r[...] + c1[...])
    e1c = _silu(prec).astype(bf16)
    cc = _silu(jnp.dot(e1c, w2c[...].astype(bf16), preferred_element_type=f32)
               + c2[...])
    c = cc[:, 0:1]
    bb = (l1 // XW == l0 % 8).astype(f32)
    lxt = (lax.broadcasted_iota(jnp.int32, (_BG, _EB), 1) // 8
           == lax.broadcasted_iota(jnp.int32, (_BG, _EB), 0)).astype(f32)
    cg = jnp.dot(lxt, c * bb, preferred_element_type=f32)
    mh_ref[...] = mh
    mxg_ref[...] = cg * xng


def _edge_mlp(ep, hs, hd, xsg, xdg, ws):
    eb = lambda i: (i, 0)
    wb = lambda i: (0, 0)
    return pl.pallas_call(
        _edge_mlp_body,
        grid=(ep // _EB,),
        in_specs=[
            pl.BlockSpec((_EB, D), eb), pl.BlockSpec((_EB, D), eb),
            pl.BlockSpec((_BG, D), eb), pl.BlockSpec((_BG, D), eb),
            pl.BlockSpec((D, H), wb), pl.BlockSpec((D, H), wb),
            pl.BlockSpec((1, H), wb), pl.BlockSpec((1, H), wb),
            pl.BlockSpec((H, H), wb), pl.BlockSpec((1, H), wb),
            pl.BlockSpec((D, H), wb), pl.BlockSpec((D, H), wb),
            pl.BlockSpec((1, H), wb), pl.BlockSpec((1, H), wb),
            pl.BlockSpec((H, XW), wb), pl.BlockSpec((1, XW), wb),
        ],
        out_specs=[
            pl.BlockSpec((_EB, H), eb),
            pl.BlockSpec((_BG, D), eb),
        ],
        out_shape=[
            jax.ShapeDtypeStruct((ep, H), jnp.float32),
            jax.ShapeDtypeStruct((ep // 8, D), jnp.float32),
        ],
    )(hs, hd, xsg, xdg, *ws)


@functools.partial(
    pl.kernel,
    out_type=(
        jax.ShapeDtypeStruct((NC, ACC_ROWS, D), jnp.float32),
        jax.ShapeDtypeStruct((NC, ACC_ROWS, XW), jnp.float32),
    ),
    mesh=_sc_mesh,
    compiler_params=_sc_params,
    scratch_types=[
        pltpu.VMEM((1, CB), jnp.int32),
        pltpu.VMEM((1, CB), jnp.int32),
        pltpu.VMEM((CB, D), jnp.float32),
        pltpu.VMEM((CB, D), jnp.float32),
        pltpu.VMEM((CB, XW), jnp.float32),
        pltpu.VMEM((CB, XW), jnp.float32),
        pltpu.VMEM_SHARED((ACC_ROWS, D), jnp.float32),
        pltpu.VMEM_SHARED((ACC_ROWS, XW), jnp.float32),
        pltpu.SemaphoreType.DMA,
        pltpu.SemaphoreType.DMA,
        pltpu.SemaphoreType.DMA,
        pltpu.SemaphoreType.DMA,
    ],
)
def _scatter_kernel(mh_ll, mx_ll, mh_rl, mx_rl, dsts_ll, dsts_rl, zh, zx,
                    part_h, part_x, idx_a, idx_b, hb_a, hb_b, xb_a, xb_b,
                    acc_h, acc_x, rsa, rsb, ssa, ssb):
    cid = lax.axis_index("c")
    sid = lax.axis_index("s")
    wid = sid * NC + cid
    r0 = sid * RPS
    pltpu.sync_copy(zh.at[pl.ds(r0, RPS)], acc_h.at[pl.ds(r0, RPS)])
    pltpu.sync_copy(zx.at[pl.ds(r0, RPS)], acc_x.at[pl.ds(r0, RPS)])
    plsc.subcore_barrier()

    def run(pw, dst3, mh_hbm, mx_hbm):
        base = wid * pw

        def fire_r(s, hb, xb, idxb, sem):
            pltpu.async_copy(mh_hbm.at[pl.ds((base + s) * CB, CB)], hb, sem)
            pltpu.async_copy(mx_hbm.at[pl.ds((base + s) * CB, CB)], xb, sem)
            pltpu.async_copy(dst3.at[wid, pl.ds(s, 1)], idxb, sem)

        def wait_r(hb, xb, idxb, sem):
            pltpu.make_async_copy(mh_hbm.at[pl.ds(0, CB)], hb, sem).wait()
            pltpu.make_async_copy(mx_hbm.at[pl.ds(0, CB)], xb, sem).wait()
            pltpu.make_async_copy(dst3.at[0, pl.ds(0, 1)], idxb, sem).wait()

        def do_sadd(hb, xb, idxb, sem):
            pltpu.async_copy(hb, acc_h.at[idxb.at[0]], sem, add=True)
            pltpu.async_copy(xb, acc_x.at[idxb.at[0]], sem, add=True)
            pltpu.make_async_copy(hb, acc_h.at[idxb.at[0]], sem).wait()
            pltpu.make_async_copy(xb, acc_x.at[idxb.at[0]], sem).wait()

        fire_r(0, hb_a, xb_a, idx_a, rsa)
        fire_r(1, hb_b, xb_b, idx_b, rsb)

        def body(g, carry):
            wait_r(hb_a, xb_a, idx_a, rsa)
            do_sadd(hb_a, xb_a, idx_a, ssa)

            @pl.when(g < pw // 2 - 1)
            def _():
                fire_r(2 * g + 2, hb_a, xb_a, idx_a, rsa)

            wait_r(hb_b, xb_b, idx_b, rsb)
            do_sadd(hb_b, xb_b, idx_b, ssb)

            @pl.when(g < pw // 2 - 1)
            def _():
                fire_r(2 * g + 3, hb_b, xb_b, idx_b, rsb)

            return carry

        lax.fori_loop(0, pw // 2, body, 0)

    run(PW_LL, dsts_ll, mh_ll, mx_ll)
    run(PW_RL, dsts_rl, mh_rl, mx_rl)
    plsc.subcore_barrier()
    pltpu.sync_copy(acc_h.at[pl.ds(r0, RPS)], part_h.at[cid, pl.ds(r0, RPS)])
    pltpu.sync_copy(acc_x.at[pl.ds(r0, RPS)], part_x.at[cid, pl.ds(r0, RPS)])


_NB = 1000


def _node_body(h_ref, ph0, ph1, xl_ref, px0, px1,
               wn1a, wn1b, bn1, wn2, bn2, ho_ref, xo_ref):
    h = h_ref[...]
    hn = ph0[...] + ph1[...]
    pre = (jnp.dot(h, wn1a[...], preferred_element_type=jnp.float32)
           + jnp.dot(hn, wn1b[...], preferred_element_type=jnp.float32)
           + bn1[...])
    m = jnp.dot(_silu(pre), wn2[...], preferred_element_type=jnp.float32) + bn2[...]
    ho_ref[...] = h + m
    xo_ref[...] = xl_ref[...] + px0[...] + px1[...]


def _node_call(h_lig, ph0, ph1, xl, px0, px1, wn1a, wn1b, bn1, wn2, bn2):
    nb = lambda i: (i, 0)
    wb = lambda i: (0, 0)
    return pl.pallas_call(
        _node_body,
        grid=(N_LIG // _NB,),
        in_specs=[
            pl.BlockSpec((_NB, D), nb), pl.BlockSpec((_NB, D), nb),
            pl.BlockSpec((_NB, D), nb),
            pl.BlockSpec((_NB, XW), nb), pl.BlockSpec((_NB, XW), nb),
            pl.BlockSpec((_NB, XW), nb),
            pl.BlockSpec((D, H), wb), pl.BlockSpec((D, H), wb),
            pl.BlockSpec((1, H), wb), pl.BlockSpec((H, D), wb),
            pl.BlockSpec((1, D), wb),
        ],
        out_specs=[
            pl.BlockSpec((_NB, D), nb),
            pl.BlockSpec((_NB, XW), nb),
        ],
        out_shape=[
            jax.ShapeDtypeStruct((N_LIG, D), jnp.float32),
            jax.ShapeDtypeStruct((N_LIG, XW), jnp.float32),
        ],
    )(h_lig, ph0, ph1, xl, px0, px1, wn1a, wn1b, bn1, wn2, bn2)


def _prep_idx(ei, ep):
    e = ei.shape[1]
    src = jnp.pad(ei[0], (0, ep - e))
    dst_g = jnp.pad(ei[1], (0, ep - e))
    dst_s = jnp.pad(ei[1], (0, ep - e), constant_values=N_LIG)
    return (src.reshape(-1, CB), dst_g.reshape(-1, CB),
            dst_s.reshape(NW, -1, CB))


def kernel(h_lig, h_rec, x_lig, x_rec, edge_index_ll, edge_index_rl,
           W1e_ll, b1e_ll, W2e_ll, b2e_ll, W1c_ll, b1c_ll, W2c_ll, b2c_ll,
           W1e_rl, b1e_rl, W2e_rl, b2e_rl, W1c_rl, b1c_rl, W2c_rl, b2c_rl,
           Wn1, bn1, Wn2, bn2):
    xl = jnp.pad(x_lig, ((0, 0), (0, XW - 3)))
    xr = jnp.pad(x_rec, ((0, 0), (0, XW - 3)))
    src_ll, dstg_ll, dsts_ll = _prep_idx(edge_index_ll, EP_LL)
    src_rl, dstg_rl, dsts_rl = _prep_idx(edge_index_rl, EP_RL)

    (hs_ll, hd_ll, xs_ll, xd_ll,
     hs_rl, hd_rl, xs_rl, xd_rl) = _gather_kernel(
        h_lig, h_rec, xl, xr, src_ll, dstg_ll, src_rl, dstg_rl)

    def ws(W1e, b1e, W2e, b2e, W1c, b1c, W2c, b2c):
        return (W1e[:D], W1e[D:2 * D], W1e[2 * D:], b1e.reshape(1, H),
                W2e, b2e.reshape(1, H),
                W1c[:D], W1c[D:2 * D], W1c[2 * D:], b1c.reshape(1, H),
                jnp.pad(W2c, ((0, 0), (0, XW - 1))),
                jnp.pad(b2c, (0, XW - 1)).reshape(1, XW))

    mh_ll, mxg_ll = _edge_mlp(EP_LL, hs_ll, hd_ll,
                              xs_ll.reshape(-1, D), xd_ll.reshape(-1, D),
                              ws(W1e_ll, b1e_ll, W2e_ll, b2e_ll,
                                 W1c_ll, b1c_ll, W2c_ll, b2c_ll))
    mh_rl, mxg_rl = _edge_mlp(EP_RL, hs_rl, hd_rl,
                              xs_rl.reshape(-1, D), xd_rl.reshape(-1, D),
                              ws(W1e_rl, b1e_rl, W2e_rl, b2e_rl,
                                 W1c_rl, b1c_rl, W2c_rl, b2c_rl))
    mx_ll = mxg_ll.reshape(-1, XW)
    mx_rl = mxg_rl.reshape(-1, XW)

    zh = jnp.zeros((ACC_ROWS, D), jnp.float32)
    zx = jnp.zeros((ACC_ROWS, XW), jnp.float32)
    part_h, part_x = _scatter_kernel(mh_ll, mx_ll, mh_rl, mx_rl,
                                     dsts_ll, dsts_rl, zh, zx)

    h_out, xo = _node_call(h_lig, part_h[0], part_h[1], xl,
                           part_x[0], part_x[1],
                           Wn1[:D], Wn1[D:], bn1.reshape(1, H),
                           Wn2, bn2.reshape(1, D))
    return (h_out, h_rec, xo[:, :3], x_rec)
